# baseline (speedup 1.0000x reference)
import functools

import jax
import jax.numpy as jnp
from jax import lax
from jax.experimental import pallas as pl
from jax.experimental.pallas import tpu as pltpu
from jax.experimental.pallas import tpu_sc as plsc

N_NODES = 10000
N_EDGES = 320000
EDGE_DIM = 16
NODE_DIM = 128

NC = 2
NS = 16
NPAD = 10240
CHUNK = 16000


def _sc_scatter_body(n_edges, ef_t_hbm, ei_hbm, zeros_hbm, part_hbm,
                     acc_v, val_v, idx_v, sems):
    c = lax.axis_index("c")
    s = lax.axis_index("s")
    e_half = n_edges // NC
    nchunk = e_half // CHUNK
    base = c * e_half
    pltpu.sync_copy(zeros_hbm, acc_v)

    def start(k, slot):
        off = base + k * CHUNK
        cp_i = pltpu.make_async_copy(ei_hbm.at[1, pl.ds(off, CHUNK)],
                                     idx_v.at[slot], sems.at[slot, 0])
        cp_i.start()
        cp_v = pltpu.make_async_copy(ef_t_hbm.at[s, pl.ds(off, CHUNK)],
                                     val_v.at[slot], sems.at[slot, 1])
        cp_v.start()

    def wait(k, slot):
        off = base + k * CHUNK
        pltpu.make_async_copy(ei_hbm.at[1, pl.ds(off, CHUNK)],
                              idx_v.at[slot], sems.at[slot, 0]).wait()
        pltpu.make_async_copy(ef_t_hbm.at[s, pl.ds(off, CHUNK)],
                              val_v.at[slot], sems.at[slot, 1]).wait()

    start(0, 0)
    for k in range(nchunk):
        slot = k % 2
        if k + 1 < nchunk:
            start(k + 1, (k + 1) % 2)
        wait(k, slot)

        @plsc.parallel_loop(0, CHUNK, step=16, unroll=32)
        def _(e):
            idx16 = idx_v[slot, pl.ds(e, 16)]
            v16 = val_v[slot, pl.ds(e, 16)]
            plsc.addupdate_scatter(acc_v, [idx16], v16)

    pltpu.sync_copy(acc_v, part_hbm.at[c, s])


def _sc_scatter(ef_t, edge_index, zeros):
    n_edges = ef_t.shape[1]
    mesh = plsc.VectorSubcoreMesh(core_axis_name="c", subcore_axis_name="s")
    return pl.kernel(
        functools.partial(_sc_scatter_body, n_edges),
        out_type=jax.ShapeDtypeStruct((NC, NS, NPAD), jnp.float32),
        mesh=mesh,
        scratch_types=[
            pltpu.VMEM((NPAD,), jnp.float32),
            pltpu.VMEM((2, CHUNK), jnp.float32),
            pltpu.VMEM((2, CHUNK), jnp.int32),
            pltpu.SemaphoreType.DMA((2, 2)),
        ],
        compiler_params=pltpu.CompilerParams(use_tc_tiling_on_sc=False,
                                             needs_layout_passes=False),
    )(ef_t, edge_index, zeros)


BLK = 2560


def _tc_mlp_body(nf_ref, p0_ref, p1_ref, w1a_ref, w1b_ref, b1_ref,
                 w2_ref, b2_ref, w3_ref, b3_ref, g_ref, be_ref, out_ref):
    agg_t = p0_ref[...] + p1_ref[...]
    x = jnp.dot(nf_ref[...], w1a_ref[...], preferred_element_type=jnp.float32)
    x = x + lax.dot_general(agg_t, w1b_ref[...], (((0,), (0,)), ((), ())),
                            preferred_element_type=jnp.float32)
    h = jnp.maximum(x + b1_ref[...], 0.0)
    h = jnp.dot(h, w2_ref[...], preferred_element_type=jnp.float32)
    h = jnp.maximum(h + b2_ref[...], 0.0)
    h = jnp.dot(h, w3_ref[...], preferred_element_type=jnp.float32)
    h = h + b3_ref[...]
    mean = jnp.mean(h, axis=-1, keepdims=True)
    var = jnp.mean((h - mean) ** 2, axis=-1, keepdims=True)
    out_ref[...] = ((h - mean) * lax.rsqrt(var + 1e-5)) * g_ref[...] + be_ref[...]


def _tc_mlp(nf, p0, p1, w1a, w1b, b1, w2, b2, w3, b3, gamma, beta):
    n = nf.shape[0]
    grid = (n + BLK - 1) // BLK
    full = lambda a: pl.BlockSpec(a.shape, lambda i: (0,) * a.ndim)
    pblk = pl.BlockSpec((EDGE_DIM, BLK), lambda i: (0, i))
    return pl.pallas_call(
        _tc_mlp_body,
        grid=(grid,),
        in_specs=[
            pl.BlockSpec((BLK, NODE_DIM), lambda i: (i, 0)),
            pblk, pblk,
            full(w1a), full(w1b), full(b1), full(w2), full(b2),
            full(w3), full(b3), full(gamma), full(beta),
        ],
        out_specs=pl.BlockSpec((BLK, NODE_DIM), lambda i: (i, 0)),
        out_shape=jax.ShapeDtypeStruct((n, NODE_DIM), jnp.float32),
    )(nf, p0, p1, w1a, w1b, b1, w2, b2, w3, b3, gamma, beta)


def kernel(node_feat, edge_feat, edge_index, n_nodes,
           W1, b1, W2, b2, W3, b3, gamma, beta):
    ef_t = edge_feat.T
    zeros = jnp.zeros((NPAD,), jnp.float32)
    parts = _sc_scatter(ef_t, edge_index, zeros)
    nz = jnp.asarray(n_nodes, jnp.float32) * 0
    beta = beta + nz
    w1a = W1[:NODE_DIM]
    w1b = W1[NODE_DIM:]
    return _tc_mlp(node_feat, parts[0], parts[1], w1a, w1b,
                   b1.reshape(1, -1), W2, b2.reshape(1, -1),
                   W3, b3.reshape(1, -1), gamma.reshape(1, -1),
                   beta.reshape(1, -1))


# speedup vs baseline: 1.0135x; 1.0135x over previous
import functools

import jax
import jax.numpy as jnp
from jax import lax
from jax.experimental import pallas as pl
from jax.experimental.pallas import tpu as pltpu
from jax.experimental.pallas import tpu_sc as plsc

N_NODES = 10000
N_EDGES = 320000
EDGE_DIM = 16
NODE_DIM = 128

NC = 2
NS = 16
NPAD = 10240
CHUNK = 16000


def _sc_scatter_body(n_edges, ef_t_hbm, ei_hbm, zeros_hbm, part_hbm,
                     acc_v, val_v, idx_v, sems):
    c = lax.axis_index("c")
    s = lax.axis_index("s")
    e_half = n_edges // NC
    nchunk = e_half // CHUNK
    base = c * e_half
    pltpu.sync_copy(zeros_hbm, acc_v)

    def start(k, slot):
        off = base + k * CHUNK
        cp_i = pltpu.make_async_copy(ei_hbm.at[1, pl.ds(off, CHUNK)],
                                     idx_v.at[slot], sems.at[slot, 0])
        cp_i.start()
        cp_v = pltpu.make_async_copy(ef_t_hbm.at[s, pl.ds(off, CHUNK)],
                                     val_v.at[slot], sems.at[slot, 1])
        cp_v.start()

    def wait(k, slot):
        off = base + k * CHUNK
        pltpu.make_async_copy(ei_hbm.at[1, pl.ds(off, CHUNK)],
                              idx_v.at[slot], sems.at[slot, 0]).wait()
        pltpu.make_async_copy(ef_t_hbm.at[s, pl.ds(off, CHUNK)],
                              val_v.at[slot], sems.at[slot, 1]).wait()

    start(0, 0)
    for k in range(nchunk):
        slot = k % 2
        if k + 1 < nchunk:
            start(k + 1, (k + 1) % 2)
        wait(k, slot)

        @plsc.parallel_loop(0, CHUNK, step=16, unroll=16)
        def _(e):
            idx16 = idx_v[slot, pl.ds(e, 16)]
            v16 = val_v[slot, pl.ds(e, 16)]
            plsc.addupdate_scatter(acc_v, [idx16], v16)

    pltpu.sync_copy(acc_v, part_hbm.at[c, s])


def _sc_scatter(ef_t, edge_index, zeros):
    n_edges = ef_t.shape[1]
    mesh = plsc.VectorSubcoreMesh(core_axis_name="c", subcore_axis_name="s")
    return pl.kernel(
        functools.partial(_sc_scatter_body, n_edges),
        out_type=jax.ShapeDtypeStruct((NC, NS, NPAD), jnp.float32),
        mesh=mesh,
        scratch_types=[
            pltpu.VMEM((NPAD,), jnp.float32),
            pltpu.VMEM((2, CHUNK), jnp.float32),
            pltpu.VMEM((2, CHUNK), jnp.int32),
            pltpu.SemaphoreType.DMA((2, 2)),
        ],
        compiler_params=pltpu.CompilerParams(use_tc_tiling_on_sc=False,
                                             needs_layout_passes=False),
    )(ef_t, edge_index, zeros)


BLK = 2560


def _tc_mlp_body(nf_ref, p0_ref, p1_ref, w1a_ref, w1b_ref, b1_ref,
                 w2_ref, b2_ref, w3_ref, b3_ref, g_ref, be_ref, out_ref):
    agg_t = p0_ref[...] + p1_ref[...]
    x = jnp.dot(nf_ref[...], w1a_ref[...], preferred_element_type=jnp.float32)
    x = x + lax.dot_general(agg_t, w1b_ref[...], (((0,), (0,)), ((), ())),
                            preferred_element_type=jnp.float32)
    h = jnp.maximum(x + b1_ref[...], 0.0)
    h = jnp.dot(h, w2_ref[...], preferred_element_type=jnp.float32)
    h = jnp.maximum(h + b2_ref[...], 0.0)
    h = jnp.dot(h, w3_ref[...], preferred_element_type=jnp.float32)
    h = h + b3_ref[...]
    mean = jnp.mean(h, axis=-1, keepdims=True)
    var = jnp.mean((h - mean) ** 2, axis=-1, keepdims=True)
    out_ref[...] = ((h - mean) * lax.rsqrt(var + 1e-5)) * g_ref[...] + be_ref[...]


def _tc_mlp(nf, p0, p1, w1a, w1b, b1, w2, b2, w3, b3, gamma, beta):
    n = nf.shape[0]
    grid = (n + BLK - 1) // BLK
    full = lambda a: pl.BlockSpec(a.shape, lambda i: (0,) * a.ndim)
    pblk = pl.BlockSpec((EDGE_DIM, BLK), lambda i: (0, i))
    return pl.pallas_call(
        _tc_mlp_body,
        grid=(grid,),
        in_specs=[
            pl.BlockSpec((BLK, NODE_DIM), lambda i: (i, 0)),
            pblk, pblk,
            full(w1a), full(w1b), full(b1), full(w2), full(b2),
            full(w3), full(b3), full(gamma), full(beta),
        ],
        out_specs=pl.BlockSpec((BLK, NODE_DIM), lambda i: (i, 0)),
        out_shape=jax.ShapeDtypeStruct((n, NODE_DIM), jnp.float32),
    )(nf, p0, p1, w1a, w1b, b1, w2, b2, w3, b3, gamma, beta)


def kernel(node_feat, edge_feat, edge_index, n_nodes,
           W1, b1, W2, b2, W3, b3, gamma, beta):
    ef_t = edge_feat.T
    zeros = jnp.zeros((NPAD,), jnp.float32)
    parts = _sc_scatter(ef_t, edge_index, zeros)
    nz = jnp.asarray(n_nodes, jnp.float32) * 0
    beta = beta + nz
    w1a = W1[:NODE_DIM]
    w1b = W1[NODE_DIM:]
    return _tc_mlp(node_feat, parts[0], parts[1], w1a, w1b,
                   b1.reshape(1, -1), W2, b2.reshape(1, -1),
                   W3, b3.reshape(1, -1), gamma.reshape(1, -1),
                   beta.reshape(1, -1))
